# baseline (speedup 1.0000x reference)
import functools

import jax
import jax.numpy as jnp
from jax import lax
from jax.experimental import pallas as pl
from jax.experimental.pallas import tpu as pltpu
from jax.experimental.pallas import tpu_sc as plsc

D = 1024
H = 3072
NE = 8
K = 2
T = 2048
NA = T * K
RB = 128
NB = NA // RB
S_MAX = NB + NE - 1
HC = 2
HCK = H // HC
NC, NS = 2, 16
NW = NC * NS
TPW = T // NW
CH = 32
RCH = 512


def _router_compute(x_ref, wr_ref, br_ref, dest_ref, gates_ref, counts_ref,
                    loss_ref):
    x = x_ref[...]
    wr = wr_ref[...]
    logits = lax.dot_general(x, wr, (((1,), (1,)), ((), ())),
                             preferred_element_type=jnp.float32)
    logits = logits + br_ref[...]
    m = jnp.max(logits, axis=1, keepdims=True)
    ex = jnp.exp(logits - m)
    probs = ex / jnp.sum(ex, axis=1, keepdims=True)

    iota_e = lax.broadcasted_iota(jnp.int32, (T, NE), 1)
    m1 = jnp.max(probs, axis=1, keepdims=True)
    i1 = jnp.min(jnp.where(probs == m1, iota_e, NE), axis=1, keepdims=True)
    oh0 = (iota_e == i1)
    probs_x = jnp.where(oh0, -1.0, probs)
    m2 = jnp.max(probs_x, axis=1, keepdims=True)
    i2 = jnp.min(jnp.where(probs_x == m2, iota_e, NE), axis=1, keepdims=True)
    oh1 = (iota_e == i2)
    oh0f = oh0.astype(jnp.float32)
    oh1f = oh1.astype(jnp.float32)

    counts = jnp.sum(oh0f, axis=0, keepdims=True) + jnp.sum(
        oh1f, axis=0, keepdims=True)
    incl = counts
    for sh in (1, 2, 4):
        incl = incl + jnp.concatenate(
            [jnp.zeros((1, sh), jnp.float32), incl[:, :NE - sh]], axis=1)
    offs = incl - counts

    rr = lax.broadcasted_iota(jnp.int32, (RCH, RCH), 0)
    rc = lax.broadcasted_iota(jnp.int32, (RCH, RCH), 1)
    lstrict = (rr > rc).astype(jnp.float32)
    carry = jnp.zeros((1, NE), jnp.float32)
    for blk in range(NA // RCH):
        if blk < T // RCH:
            oh = oh0f[blk * RCH:(blk + 1) * RCH]
        else:
            oh = oh1f[(blk - T // RCH) * RCH:(blk - T // RCH + 1) * RCH]
        intra = lax.dot_general(lstrict, oh, (((1,), (0,)), ((), ())),
                                preferred_element_type=jnp.float32)
        dest_blk = jnp.sum(oh * (intra + carry + offs), axis=1, keepdims=True)
        dest_ref[blk * RCH:(blk + 1) * RCH, :] = dest_blk.astype(jnp.int32)
        carry = carry + jnp.sum(oh, axis=0, keepdims=True)

    gates_ref[...] = jnp.concatenate([m1, m2], axis=1)
    counts_ref[...] = counts
    pm = jnp.mean(probs, axis=0, keepdims=True)
    tf = counts / jnp.sum(counts)
    loss_ref[...] = jnp.sum(pm * tf, axis=1, keepdims=True) * NE


def _router(x2d, wr, br2d):
    return pl.pallas_call(
        _router_compute,
        out_shape=[
            jax.ShapeDtypeStruct((NA, 1), jnp.int32),
            jax.ShapeDtypeStruct((T, K), jnp.float32),
            jax.ShapeDtypeStruct((1, NE), jnp.float32),
            jax.ShapeDtypeStruct((1, 1), jnp.float32),
        ],
    )(x2d, wr, br2d)


def _schedule(counts):
    c = counts[0].astype(jnp.int32)
    off = jnp.concatenate(
        [jnp.zeros((1,), jnp.int32), jnp.cumsum(c)])
    blk_lo = (jnp.arange(NB, dtype=jnp.int32) * RB)[:, None]
    ilo = jnp.maximum(blk_lo, off[:NE][None, :])
    ihi = jnp.minimum(blk_lo + RB, off[1:][None, :])
    valid = (ihi > ilo).ravel()
    pos = jnp.cumsum(valid.astype(jnp.int32)) - 1
    tgt = jnp.where(valid, pos, S_MAX)
    bvals = jnp.repeat(jnp.arange(NB, dtype=jnp.int32), NE)
    evals = jnp.tile(jnp.arange(NE, dtype=jnp.int32), NB)
    e_last = jnp.sum(off[:NE] < NA).astype(jnp.int32) - 1
    sb = jnp.full((S_MAX,), NB - 1, jnp.int32).at[tgt].set(bvals, mode="drop")
    se = jnp.full((S_MAX,), e_last, jnp.int32).at[tgt].set(evals, mode="drop")
    lo = jnp.full((S_MAX,), NA, jnp.int32).at[tgt].set(
        ilo.ravel().astype(jnp.int32), mode="drop")
    hi = jnp.full((S_MAX,), NA, jnp.int32).at[tgt].set(
        ihi.ravel().astype(jnp.int32), mode="drop")
    return jnp.stack([sb, se, lo, hi])


def _ffn1_body(sched_ref, xs_ref, w1_ref, b1_ref, h_ref, w1b_scr):
    s = pl.program_id(0)
    b = sched_ref[0, s]
    lo = sched_ref[2, s]
    hi = sched_ref[3, s]
    rows = b * RB + lax.broadcasted_iota(jnp.int32, (RB, 1), 0)
    mask = (rows >= lo) & (rows < hi)
    e_changed = jnp.logical_or(
        s == 0, sched_ref[1, s] != sched_ref[1, jnp.maximum(s - 1, 0)])

    @pl.when(e_changed)
    def _():
        w1b_scr[...] = w1_ref[0].astype(jnp.bfloat16)

    xb = xs_ref[...].astype(jnp.bfloat16)
    hpre = lax.dot_general(xb, w1b_scr[...], (((1,), (1,)), ((), ())),
                           preferred_element_type=jnp.float32)
    hpre = hpre + b1_ref[0, 0][None, :]
    hact = 0.5 * hpre * (1.0 + lax.erf(hpre * 0.7071067811865476))
    h_ref[...] = jnp.where(mask, hact.astype(jnp.bfloat16), h_ref[...])


def _ffn2_body(sched_ref, h_ref, w2_ref, b2_ref, y_ref, w2b_scr):
    s = pl.program_id(0)
    b = sched_ref[0, s]
    lo = sched_ref[2, s]
    hi = sched_ref[3, s]
    rows = b * RB + lax.broadcasted_iota(jnp.int32, (RB, 1), 0)
    mask = (rows >= lo) & (rows < hi)
    e_changed = jnp.logical_or(
        s == 0, sched_ref[1, s] != sched_ref[1, jnp.maximum(s - 1, 0)])

    @pl.when(e_changed)
    def _():
        w2b_scr[...] = w2_ref[0].astype(jnp.bfloat16)

    yc = lax.dot_general(h_ref[...], w2b_scr[...], (((1,), (1,)), ((), ())),
                         preferred_element_type=jnp.float32)
    y_ref[...] = jnp.where(mask, yc + b2_ref[0, 0][None, :], y_ref[...])


def _ffn(sched, xs, w1, b1r, w2, b2r):
    gs1 = pltpu.PrefetchScalarGridSpec(
        num_scalar_prefetch=1,
        grid=(S_MAX,),
        in_specs=[
            pl.BlockSpec((RB, D), lambda s, sref: (sref[0, s], 0)),
            pl.BlockSpec((1, H, D), lambda s, sref: (sref[1, s], 0, 0)),
            pl.BlockSpec((1, 1, H), lambda s, sref: (sref[1, s], 0, 0)),
        ],
        out_specs=pl.BlockSpec((RB, H), lambda s, sref: (sref[0, s], 0)),
        scratch_shapes=[pltpu.VMEM((H, D), jnp.bfloat16)],
    )
    h = pl.pallas_call(
        _ffn1_body,
        grid_spec=gs1,
        out_shape=jax.ShapeDtypeStruct((NA, H), jnp.bfloat16),
        compiler_params=pltpu.CompilerParams(
            dimension_semantics=("arbitrary",)),
    )(sched, xs, w1, b1r)
    gs2 = pltpu.PrefetchScalarGridSpec(
        num_scalar_prefetch=1,
        grid=(S_MAX,),
        in_specs=[
            pl.BlockSpec((RB, H), lambda s, sref: (sref[0, s], 0)),
            pl.BlockSpec((1, D, H), lambda s, sref: (sref[1, s], 0, 0)),
            pl.BlockSpec((1, 1, D), lambda s, sref: (sref[1, s], 0, 0)),
        ],
        out_specs=pl.BlockSpec((RB, D), lambda s, sref: (sref[0, s], 0)),
        scratch_shapes=[pltpu.VMEM((D, H), jnp.bfloat16)],
    )
    return pl.pallas_call(
        _ffn2_body,
        grid_spec=gs2,
        out_shape=jax.ShapeDtypeStruct((NA, D), jnp.float32),
        compiler_params=pltpu.CompilerParams(
            dimension_semantics=("arbitrary",)),
    )(sched, h, w2, b2r)


def _dispatch(x2d, destp64):
    mesh = plsc.VectorSubcoreMesh(core_axis_name="c", subcore_axis_name="s")

    @functools.partial(
        pl.kernel,
        out_type=jax.ShapeDtypeStruct((NA, D), jnp.float32),
        mesh=mesh,
        scratch_types=[
            pltpu.VMEM((TPW,), jnp.int32),
            pltpu.VMEM((TPW,), jnp.int32),
            pltpu.VMEM((TPW, D), jnp.float32),
            pltpu.SemaphoreType.DMA,
            pltpu.SemaphoreType.DMA,
        ],
    )
    def run(x_hbm, dp_hbm, xs_hbm, idx0_v, idx1_v, x_v, sem0, sem1):
        wid = lax.axis_index("s") * NC + lax.axis_index("c")
        base = wid * TPW
        pltpu.sync_copy(dp_hbm.at[wid, 0], idx0_v)
        pltpu.sync_copy(dp_hbm.at[wid, 1], idx1_v)
        pltpu.sync_copy(x_hbm.at[pl.ds(base, TPW)], x_v)
        c0 = pltpu.async_copy(x_v, xs_hbm.at[idx0_v], sem0)
        c1 = pltpu.async_copy(x_v, xs_hbm.at[idx1_v], sem1)
        c0.wait()
        c1.wait()

    return run(x2d, destp64)


def _combine(y, destp32, gp32):
    mesh = plsc.VectorSubcoreMesh(core_axis_name="c", subcore_axis_name="s")

    @functools.partial(
        pl.kernel,
        out_type=jax.ShapeDtypeStruct((T, D), jnp.float32),
        mesh=mesh,
        scratch_types=[
            pltpu.VMEM((K, TPW // CH, CH), jnp.int32),
            pltpu.VMEM((K, TPW, 16), jnp.float32),
            pltpu.VMEM((CH, D), jnp.float32),
            pltpu.VMEM((CH, D), jnp.float32),
            pltpu.SemaphoreType.DMA,
            pltpu.SemaphoreType.DMA,
        ],
    )
    def run(y_hbm, dp_hbm, gp_hbm, out_hbm, idx_v, g_v, r0, r1, sem0, sem1):
        wid = lax.axis_index("s") * NC + lax.axis_index("c")
        base = wid * TPW
        pltpu.sync_copy(dp_hbm.at[wid], idx_v)
        pltpu.sync_copy(gp_hbm.at[wid], g_v)
        for c in range(TPW // CH):
            ca = pltpu.async_copy(y_hbm.at[idx_v.at[0, c]], r0, sem0)
            cb = pltpu.async_copy(y_hbm.at[idx_v.at[1, c]], r1, sem1)
            ca.wait()
            cb.wait()

            def row_body(r, _):
                g0 = g_v[0, c * CH + r]
                g1 = g_v[1, c * CH + r]

                def col_body(j, _):
                    col = j * 16
                    a = r0[r, pl.ds(col, 16)]
                    bv = r1[r, pl.ds(col, 16)]
                    r0[r, pl.ds(col, 16)] = g0 * a + g1 * bv
                    return 0

                lax.fori_loop(0, D // 16, col_body, 0, unroll=4)
                return 0

            lax.fori_loop(0, CH, row_body, 0)
            pltpu.sync_copy(r0, out_hbm.at[pl.ds(base + c * CH, CH)])

    return run(y, destp32, gp32)


def kernel(x, Wr, br, W1, b1, W2, b2):
    x2d = x.reshape(T, D)
    dest_out, gates, counts, loss = _router(x2d, Wr, br.reshape(1, NE))
    dest = dest_out[:, 0]
    destp64 = dest.reshape(K, NW, TPW).transpose(1, 0, 2)
    destp32 = dest.reshape(K, NW, TPW // CH, CH).transpose(1, 0, 2, 3)
    gp32 = jnp.broadcast_to(
        gates.T.reshape(K, NW, TPW).transpose(1, 0, 2)[..., None],
        (NW, K, TPW, 16))
    sched = _schedule(counts)
    xs = _dispatch(x2d, destp64)
    y = _ffn(sched, xs, W1, b1.reshape(NE, 1, H), W2, b2.reshape(NE, 1, D))
    out = _combine(y, destp32, gp32)
    return out.reshape(1, T, D), loss[0, 0]


# speedup vs baseline: 1.2881x; 1.2881x over previous
import functools

import jax
import jax.numpy as jnp
from jax import lax
from jax.experimental import pallas as pl
from jax.experimental.pallas import tpu as pltpu
from jax.experimental.pallas import tpu_sc as plsc

D = 1024
H = 3072
NE = 8
K = 2
T = 2048
NA = T * K
RB = 256
NB = NA // RB
S_MAX = NB + NE - 1
HC = 2
HCK = H // HC
NC, NS = 2, 16
NW = NC * NS
TPW = T // NW
CH = 32
RCH = 128


def _router_compute(x_ref, wr_ref, br_ref, dest_ref, gates_ref, counts_ref,
                    loss_ref):
    x = x_ref[...]
    wr = wr_ref[...]
    logits = lax.dot_general(x, wr, (((1,), (1,)), ((), ())),
                             preferred_element_type=jnp.float32)
    logits = logits + br_ref[...]
    m = jnp.max(logits, axis=1, keepdims=True)
    ex = jnp.exp(logits - m)
    probs = ex / jnp.sum(ex, axis=1, keepdims=True)

    iota_e = lax.broadcasted_iota(jnp.int32, (T, NE), 1)
    m1 = jnp.max(probs, axis=1, keepdims=True)
    i1 = jnp.min(jnp.where(probs == m1, iota_e, NE), axis=1, keepdims=True)
    oh0 = (iota_e == i1)
    probs_x = jnp.where(oh0, -1.0, probs)
    m2 = jnp.max(probs_x, axis=1, keepdims=True)
    i2 = jnp.min(jnp.where(probs_x == m2, iota_e, NE), axis=1, keepdims=True)
    oh1 = (iota_e == i2)
    oh0f = oh0.astype(jnp.float32)
    oh1f = oh1.astype(jnp.float32)

    counts = jnp.sum(oh0f, axis=0, keepdims=True) + jnp.sum(
        oh1f, axis=0, keepdims=True)
    incl = counts
    for sh in (1, 2, 4):
        incl = incl + jnp.concatenate(
            [jnp.zeros((1, sh), jnp.float32), incl[:, :NE - sh]], axis=1)
    offs = incl - counts

    rr = lax.broadcasted_iota(jnp.int32, (RCH, RCH), 0)
    rc = lax.broadcasted_iota(jnp.int32, (RCH, RCH), 1)
    lstrict = (rr > rc).astype(jnp.float32)
    carry = jnp.zeros((1, NE), jnp.float32)
    for blk in range(NA // RCH):
        if blk < T // RCH:
            oh = oh0f[blk * RCH:(blk + 1) * RCH]
        else:
            oh = oh1f[(blk - T // RCH) * RCH:(blk - T // RCH + 1) * RCH]
        intra = lax.dot_general(lstrict, oh, (((1,), (0,)), ((), ())),
                                preferred_element_type=jnp.float32)
        dest_blk = jnp.sum(oh * (intra + carry + offs), axis=1, keepdims=True)
        dest_ref[blk * RCH:(blk + 1) * RCH, :] = dest_blk.astype(jnp.int32)
        carry = carry + jnp.sum(oh, axis=0, keepdims=True)

    gates_ref[...] = jnp.concatenate([m1, m2], axis=1)
    counts_ref[...] = counts
    pm = jnp.mean(probs, axis=0, keepdims=True)
    tf = counts / jnp.sum(counts)
    loss_ref[...] = jnp.sum(pm * tf, axis=1, keepdims=True) * NE


def _router(x2d, wr, br2d):
    return pl.pallas_call(
        _router_compute,
        out_shape=[
            jax.ShapeDtypeStruct((NA, 1), jnp.int32),
            jax.ShapeDtypeStruct((T, K), jnp.float32),
            jax.ShapeDtypeStruct((1, NE), jnp.float32),
            jax.ShapeDtypeStruct((1, 1), jnp.float32),
        ],
    )(x2d, wr, br2d)


def _schedule(counts):
    c = counts[0].astype(jnp.int32)
    off = jnp.concatenate(
        [jnp.zeros((1,), jnp.int32), jnp.cumsum(c)])
    blk_lo = (jnp.arange(NB, dtype=jnp.int32) * RB)[:, None]
    ilo = jnp.maximum(blk_lo, off[:NE][None, :])
    ihi = jnp.minimum(blk_lo + RB, off[1:][None, :])
    valid = (ihi > ilo).ravel()
    pos = jnp.cumsum(valid.astype(jnp.int32)) - 1
    tgt = jnp.where(valid, pos, S_MAX)
    bvals = jnp.repeat(jnp.arange(NB, dtype=jnp.int32), NE)
    evals = jnp.tile(jnp.arange(NE, dtype=jnp.int32), NB)
    e_last = jnp.sum(off[:NE] < NA).astype(jnp.int32) - 1
    sb = jnp.full((S_MAX,), NB - 1, jnp.int32).at[tgt].set(bvals, mode="drop")
    se = jnp.full((S_MAX,), e_last, jnp.int32).at[tgt].set(evals, mode="drop")
    lo = jnp.full((S_MAX,), NA, jnp.int32).at[tgt].set(
        ilo.ravel().astype(jnp.int32), mode="drop")
    hi = jnp.full((S_MAX,), NA, jnp.int32).at[tgt].set(
        ihi.ravel().astype(jnp.int32), mode="drop")
    return jnp.stack([sb, se, lo, hi])


def _ffn_body(sched_ref, xs_ref, w1_ref, b1_ref, w2_ref, b2_ref, y_ref):
    s = pl.program_id(0)
    h = pl.program_id(1)
    b = sched_ref[0, s]
    lo = sched_ref[2, s]
    hi = sched_ref[3, s]
    rows = b * RB + lax.broadcasted_iota(jnp.int32, (RB, 1), 0)
    mask = (rows >= lo) & (rows < hi)

    xb = xs_ref[...].astype(jnp.bfloat16)
    w1 = w1_ref[0].astype(jnp.bfloat16)
    hpre = lax.dot_general(xb, w1, (((1,), (1,)), ((), ())),
                           preferred_element_type=jnp.float32)
    hpre = hpre + b1_ref[0, 0][None, :]
    hact = 0.5 * hpre * (1.0 + lax.erf(hpre * 0.7071067811865476))
    hb = hact.astype(jnp.bfloat16)
    w2 = w2_ref[0].astype(jnp.bfloat16)
    yc = lax.dot_general(hb, w2, (((1,), (1,)), ((), ())),
                         preferred_element_type=jnp.float32)

    @pl.when(h == 0)
    def _():
        y_ref[...] = jnp.where(mask, yc + b2_ref[0, 0][None, :], y_ref[...])

    @pl.when(h != 0)
    def _():
        y_ref[...] = y_ref[...] + jnp.where(mask, yc, 0.0)


def _ffn(sched, xs, w1, b1r, w2, b2r):
    grid_spec = pltpu.PrefetchScalarGridSpec(
        num_scalar_prefetch=1,
        grid=(S_MAX, HC),
        in_specs=[
            pl.BlockSpec((RB, D), lambda s, h, sref: (sref[0, s], 0)),
            pl.BlockSpec((1, HCK, D), lambda s, h, sref: (sref[1, s], h, 0)),
            pl.BlockSpec((1, 1, HCK), lambda s, h, sref: (sref[1, s], 0, h)),
            pl.BlockSpec((1, D, HCK), lambda s, h, sref: (sref[1, s], 0, h)),
            pl.BlockSpec((1, 1, D), lambda s, h, sref: (sref[1, s], 0, 0)),
        ],
        out_specs=pl.BlockSpec((RB, D), lambda s, h, sref: (sref[0, s], 0)),
    )
    return pl.pallas_call(
        _ffn_body,
        grid_spec=grid_spec,
        out_shape=jax.ShapeDtypeStruct((NA, D), jnp.float32),
        compiler_params=pltpu.CompilerParams(
            dimension_semantics=("arbitrary", "arbitrary")),
    )(sched, xs, w1, b1r, w2, b2r)


def _dispatch(x2d, destp64):
    mesh = plsc.VectorSubcoreMesh(core_axis_name="c", subcore_axis_name="s")

    @functools.partial(
        pl.kernel,
        out_type=jax.ShapeDtypeStruct((NA, D), jnp.float32),
        mesh=mesh,
        scratch_types=[
            pltpu.VMEM((TPW,), jnp.int32),
            pltpu.VMEM((TPW,), jnp.int32),
            pltpu.VMEM((TPW, D), jnp.float32),
            pltpu.SemaphoreType.DMA,
            pltpu.SemaphoreType.DMA,
        ],
    )
    def run(x_hbm, dp_hbm, xs_hbm, idx0_v, idx1_v, x_v, sem0, sem1):
        wid = lax.axis_index("s") * NC + lax.axis_index("c")
        base = wid * TPW
        pltpu.sync_copy(dp_hbm.at[wid, 0], idx0_v)
        pltpu.sync_copy(dp_hbm.at[wid, 1], idx1_v)
        pltpu.sync_copy(x_hbm.at[pl.ds(base, TPW)], x_v)
        c0 = pltpu.async_copy(x_v, xs_hbm.at[idx0_v], sem0)
        c1 = pltpu.async_copy(x_v, xs_hbm.at[idx1_v], sem1)
        c0.wait()
        c1.wait()

    return run(x2d, destp64)


def _combine(y, destp32, gp32):
    mesh = plsc.VectorSubcoreMesh(core_axis_name="c", subcore_axis_name="s")

    @functools.partial(
        pl.kernel,
        out_type=jax.ShapeDtypeStruct((T, D), jnp.float32),
        mesh=mesh,
        scratch_types=[
            pltpu.VMEM((K, TPW // CH, CH), jnp.int32),
            pltpu.VMEM((K, TPW, 16), jnp.float32),
            pltpu.VMEM((CH, D), jnp.float32),
            pltpu.VMEM((CH, D), jnp.float32),
            pltpu.SemaphoreType.DMA,
            pltpu.SemaphoreType.DMA,
        ],
    )
    def run(y_hbm, dp_hbm, gp_hbm, out_hbm, idx_v, g_v, r0, r1, sem0, sem1):
        wid = lax.axis_index("s") * NC + lax.axis_index("c")
        base = wid * TPW
        pltpu.sync_copy(dp_hbm.at[wid], idx_v)
        pltpu.sync_copy(gp_hbm.at[wid], g_v)
        for c in range(TPW // CH):
            ca = pltpu.async_copy(y_hbm.at[idx_v.at[0, c]], r0, sem0)
            cb = pltpu.async_copy(y_hbm.at[idx_v.at[1, c]], r1, sem1)
            ca.wait()
            cb.wait()

            def row_body(r, _):
                g0 = g_v[0, c * CH + r]
                g1 = g_v[1, c * CH + r]

                def col_body(j, _):
                    col = j * 16
                    a = r0[r, pl.ds(col, 16)]
                    bv = r1[r, pl.ds(col, 16)]
                    r0[r, pl.ds(col, 16)] = g0 * a + g1 * bv
                    return 0

                lax.fori_loop(0, D // 16, col_body, 0, unroll=4)
                return 0

            lax.fori_loop(0, CH, row_body, 0)
            pltpu.sync_copy(r0, out_hbm.at[pl.ds(base + c * CH, CH)])

    return run(y, destp32, gp32)


def kernel(x, Wr, br, W1, b1, W2, b2):
    x2d = x.reshape(T, D)
    dest_out, gates, counts, loss = _router(x2d, Wr, br.reshape(1, NE))
    dest = dest_out[:, 0]
    destp64 = dest.reshape(K, NW, TPW).transpose(1, 0, 2)
    destp32 = dest.reshape(K, NW, TPW // CH, CH).transpose(1, 0, 2, 3)
    gp32 = jnp.broadcast_to(
        gates.T.reshape(K, NW, TPW).transpose(1, 0, 2)[..., None],
        (NW, K, TPW, 16))
    sched = _schedule(counts)
    xs = _dispatch(x2d, destp64)
    y = _ffn(sched, xs, W1, b1.reshape(NE, 1, H), W2, b2.reshape(NE, 1, D))
    out = _combine(y, destp32, gp32)
    return out.reshape(1, T, D), loss[0, 0]
